# Initial kernel scaffold; baseline (speedup 1.0000x reference)
import jax
import jax.numpy as jnp
from jax.experimental import pallas as pl


def kernel(output_points, target_points, n_samples):
    raise NotImplementedError("write your pallas kernel here")


# speedup vs baseline: 1.1153x; 1.1153x over previous
import jax
import jax.numpy as jnp
from jax.experimental import pallas as pl
from jax.experimental.pallas import tpu as pltpu

_N = 2048
_TILE = 256


def _pair_body(x_ref, y_ref, out_ref):
    x = x_ref[0]
    y = y_ref[0]
    y2 = jnp.sum(y * y, axis=0)
    colmin = jnp.full((_N,), jnp.inf, dtype=jnp.float32)
    rowsum = jnp.float32(0.0)
    for t in range(_N // _TILE):
        xt = x[:, t * _TILE:(t + 1) * _TILE]
        x2t = jnp.sum(xt * xt, axis=0)
        xy = jax.lax.dot_general(
            xt, y, (((0,), (0,)), ((), ())),
            preferred_element_type=jnp.float32)
        d = jnp.maximum(x2t[:, None] + y2[None, :] - 2.0 * xy, 0.0)
        rowsum += jnp.sum(jnp.min(d, axis=1))
        colmin = jnp.minimum(colmin, jnp.min(d, axis=0))
    out_ref[pl.program_id(0), 0] = rowsum / _N + jnp.sum(colmin) / _N


@jax.jit
def _chamfer_pairs(x, y):
    p = x.shape[0]
    return pl.pallas_call(
        _pair_body,
        grid=(p,),
        in_specs=[
            pl.BlockSpec((1, 8, _N), lambda i: (i, 0, 0)),
            pl.BlockSpec((1, 8, _N), lambda i: (i, 0, 0)),
        ],
        out_specs=pl.BlockSpec((p, 1), lambda i: (0, 0), memory_space=pltpu.SMEM),
        out_shape=jax.ShapeDtypeStruct((p, 1), jnp.float32),
    )(x, y)


def _coords_major(pts):
    b, s, n, _ = pts.shape
    p = pts.reshape(b * s, n, 3).transpose(0, 2, 1)
    return jnp.concatenate(
        [p, jnp.zeros((b * s, 5, n), dtype=pts.dtype)], axis=1)


def kernel(output_points, target_points, n_samples):
    b, s, n, _ = output_points.shape
    x = _coords_major(output_points)
    y = _coords_major(target_points)
    per_pair = _chamfer_pairs(x, y)[:, 0].reshape(b, s)
    tensor = per_pair.T
    means = jnp.mean(tensor, axis=1)
    return (means, tensor)
